# baseline (speedup 1.0000x reference)
import jax
import jax.numpy as jnp
from jax import lax
from jax.experimental import pallas as pl
from jax.experimental.pallas import tpu as pltpu
from jax.experimental.pallas import tpu_sc as plsc

_N = 10000
_D = 128
_E = 320000
_NC = 2
_NS = 16
_C = 128
_NCHG = _E // _C
_NCH0 = _NCHG // _NS
_XTRA = _NCHG - _NCH0 * _NS
_RPT = 640
_NR = 3
_NI = 6
_K0 = 2
_NQ = (_NCH0 + 1 - _K0 + 1 + _NI - 1) // _NI
_NQ0 = _NQ - 2


def _mm_body(xu_ref, wu_ref, xi_ref, wi_ref, eu_ref, ei_ref,
             yu_ref, yi_ref, z_ref, su_ref, du_ref, si_ref, di_ref):
    z_ref[...] = jnp.zeros_like(z_ref)
    su_ref[...] = eu_ref[0]
    du_ref[...] = eu_ref[1]
    si_ref[...] = ei_ref[0]
    di_ref[...] = ei_ref[1]
    yu_ref[...] = lax.dot_general(
        xu_ref[...], wu_ref[...], dimension_numbers=(((1,), (0,)), ((), ())),
        preferred_element_type=jnp.float32, precision=lax.Precision.DEFAULT)
    yi_ref[...] = lax.dot_general(
        xi_ref[...], wi_ref[...], dimension_numbers=(((1,), (0,)), ((), ())),
        preferred_element_type=jnp.float32, precision=lax.Precision.DEFAULT)


def _tc_matmul(x_user, W_u2i, x_item, W_i2u, e_u2i, e_i2u):
    nb = 5
    blk = _N // nb
    eblk = 65536
    return pl.pallas_call(
        _mm_body,
        grid=(nb,),
        in_specs=[
            pl.BlockSpec((blk, _D), lambda b: (b, 0)),
            pl.BlockSpec((_D, _D), lambda b: (0, 0)),
            pl.BlockSpec((blk, _D), lambda b: (b, 0)),
            pl.BlockSpec((_D, _D), lambda b: (0, 0)),
            pl.BlockSpec((2, eblk), lambda b: (0, b)),
            pl.BlockSpec((2, eblk), lambda b: (0, b)),
        ],
        out_specs=[
            pl.BlockSpec((blk, _D), lambda b: (b, 0)),
            pl.BlockSpec((blk, _D), lambda b: (b, 0)),
            pl.BlockSpec((blk, _D), lambda b: (b, 0)),
            pl.BlockSpec((eblk,), lambda b: (b,)),
            pl.BlockSpec((eblk,), lambda b: (b,)),
            pl.BlockSpec((eblk,), lambda b: (b,)),
            pl.BlockSpec((eblk,), lambda b: (b,)),
        ],
        out_shape=[
            jax.ShapeDtypeStruct((_N, _D), jnp.float32),
            jax.ShapeDtypeStruct((_N, _D), jnp.float32),
            jax.ShapeDtypeStruct((_N, _D), jnp.float32),
            jax.ShapeDtypeStruct((_E,), jnp.int32),
            jax.ShapeDtypeStruct((_E,), jnp.int32),
            jax.ShapeDtypeStruct((_E,), jnp.int32),
            jax.ShapeDtypeStruct((_E,), jnp.int32),
        ],
    )(x_user, W_u2i, x_item, W_i2u, e_u2i, e_i2u)


def _sc_body(y_u, y_i, s_u2i, d_u2i, s_i2u, d_i2u, zero_ref,
             out_item, out_user,
             sidx, didx, rows, agg,
             sem_i, sem_g, sem_s):
    c = lax.axis_index("c")
    s = lax.axis_index("s")
    rb = s * _RPT

    def run_rel(y_ref, src_ref, dst_ref, out_ref):
        cbase = _NCH0 * s + jnp.minimum(s, _XTRA)
        ncht = _NCH0 + jnp.where(s < _XTRA, 1, 0)

        def load_idx(k, b):
            eb = (cbase + k) * _C
            pltpu.async_copy(src_ref.at[pl.ds(eb, _C)], sidx[b], sem_i[b])
            pltpu.async_copy(dst_ref.at[pl.ds(eb, _C)], didx[b], sem_i[b])

        def wait_idx(b):
            pltpu.make_async_copy(
                src_ref.at[pl.ds(0, _C)], sidx[b], sem_i[b]).wait()
            pltpu.make_async_copy(
                dst_ref.at[pl.ds(0, _C)], didx[b], sem_i[b]).wait()

        def start_gather(b, g):
            pltpu.async_copy(y_ref.at[sidx[b]], rows[g], sem_g[g])

        def wait_gather(g):
            pltpu.make_async_copy(y_ref.at[sidx[0]], rows[g], sem_g[g]).wait()

        def start_scatter(g, b):
            pltpu.async_copy(rows[g], agg.at[didx[b]], sem_s[g], add=True)

        def wait_scatter(g, b):
            pltpu.make_async_copy(rows[g], agg.at[didx[b]], sem_s[g]).wait()

        for b in range(_NI):
            load_idx(b, b)
        @pl.when(s < _NS - 1)
        def _():
            pltpu.sync_copy(zero_ref.at[pl.ds(rb, _RPT)],
                            agg.at[pl.ds(rb, _RPT)])

        @pl.when(s == _NS - 1)
        def _():
            pltpu.sync_copy(zero_ref.at[pl.ds((_NS - 1) * _RPT, _N - (_NS - 1) * _RPT)],
                            agg.at[pl.ds((_NS - 1) * _RPT, _N - (_NS - 1) * _RPT)])

        plsc.subcore_barrier()
        wait_idx(0)
        start_gather(0, 0)
        wait_idx(1)
        start_gather(1, 1)
        wait_gather(0)
        start_scatter(0, 0)
        wait_idx(2)
        start_gather(2, 2)
        wait_gather(1)
        start_scatter(1, 1)
        wait_scatter(0, 0)
        load_idx(_NI, 0)
        wait_idx(3)
        start_gather(3, 0)

        def step(j, guarded):
            for u in range(_NI):
                k = _K0 + _NI * j + u
                r = (_K0 + u) % _NR
                b = (_K0 + u) % _NI

                def guard(off, fn):
                    if guarded:
                        pl.when(k <= ncht - off)(fn)
                    else:
                        fn()

                guard(1, lambda: wait_gather(r))
                guard(1, lambda: start_scatter(r, b))
                guard(0, lambda: wait_scatter((r + 2) % _NR, (b + 5) % _NI))
                guard(_NI, lambda: load_idx(k + _NI - 1, (b + 5) % _NI))
                guard(3, lambda: wait_idx((b + 2) % _NI))
                guard(3, lambda: start_gather((b + 2) % _NI, (r + 2) % _NR))

            return 0

        lax.fori_loop(0, _NQ0, lambda j, c: step(j, False), 0)
        lax.fori_loop(_NQ0, _NQ, lambda j, c: step(j, True), 0)

        plsc.subcore_barrier()
        @pl.when(s < _NS - 1)
        def _():
            pltpu.sync_copy(agg.at[pl.ds(rb, _RPT)],
                            out_ref.at[pl.ds(rb, _RPT)])

        @pl.when(s == _NS - 1)
        def _():
            pltpu.sync_copy(agg.at[pl.ds((_NS - 1) * _RPT, _N - (_NS - 1) * _RPT)],
                            out_ref.at[pl.ds((_NS - 1) * _RPT, _N - (_NS - 1) * _RPT)])
    @pl.when(c == 0)
    def _():
        run_rel(y_u, s_u2i, d_u2i, out_item)

    @pl.when(c == 1)
    def _():
        run_rel(y_i, s_i2u, d_i2u, out_user)


_sc_scatter = pl.kernel(
    _sc_body,
    out_type=(
        jax.ShapeDtypeStruct((_N, _D), jnp.float32),
        jax.ShapeDtypeStruct((_N, _D), jnp.float32),
    ),
    mesh=plsc.VectorSubcoreMesh(core_axis_name="c", subcore_axis_name="s",
                                num_cores=_NC, num_subcores=_NS),
    scratch_types=[
        [pltpu.VMEM((_C,), jnp.int32) for _ in range(_NI)],
        [pltpu.VMEM((_C,), jnp.int32) for _ in range(_NI)],
        [pltpu.VMEM((_C, _D), jnp.float32) for _ in range(_NR)],
        pltpu.VMEM_SHARED((_N, _D), jnp.float32),
        [pltpu.SemaphoreType.DMA for _ in range(_NI)],
        [pltpu.SemaphoreType.DMA for _ in range(_NR)],
        [pltpu.SemaphoreType.DMA for _ in range(_NR)],
    ],
)


def kernel(x_user, x_item, edge_index_u2i, edge_index_i2u, W_u2i, W_i2u):
    y_user, y_item, zeros, su, du, si, di = _tc_matmul(
        x_user, W_u2i, x_item, W_i2u, edge_index_u2i, edge_index_i2u)
    out_item, out_user = _sc_scatter(
        y_user, y_item, su, du, si, di, zeros)
    return (out_user, out_item)


# speedup vs baseline: 1.0184x; 1.0184x over previous
import jax
import jax.numpy as jnp
from jax import lax
from jax.experimental import pallas as pl
from jax.experimental.pallas import tpu as pltpu
from jax.experimental.pallas import tpu_sc as plsc

_N = 10000
_D = 128
_E = 320000
_NC = 2
_NS = 16
_C = 104
_EPT = _E // _NS
_NCH = _EPT // _C
_CT = _EPT - _NCH * _C
_RPT = 640
_NR = 3
_NI = 6
_K0 = 2
_NQ = (_NCH - _K0 + 1 + _NI - 1) // _NI


def _mm_body(xu_ref, wu_ref, xi_ref, wi_ref, eu_ref, ei_ref,
             yu_ref, yi_ref, z_ref, su_ref, du_ref, si_ref, di_ref):
    z_ref[...] = jnp.zeros_like(z_ref)
    su_ref[...] = eu_ref[0]
    du_ref[...] = eu_ref[1]
    si_ref[...] = ei_ref[0]
    di_ref[...] = ei_ref[1]
    yu_ref[...] = lax.dot_general(
        xu_ref[...], wu_ref[...], dimension_numbers=(((1,), (0,)), ((), ())),
        preferred_element_type=jnp.float32, precision=lax.Precision.DEFAULT)
    yi_ref[...] = lax.dot_general(
        xi_ref[...], wi_ref[...], dimension_numbers=(((1,), (0,)), ((), ())),
        preferred_element_type=jnp.float32, precision=lax.Precision.DEFAULT)


def _tc_matmul(x_user, W_u2i, x_item, W_i2u, e_u2i, e_i2u):
    nb = 5
    blk = _N // nb
    eblk = 65536
    return pl.pallas_call(
        _mm_body,
        grid=(nb,),
        in_specs=[
            pl.BlockSpec((blk, _D), lambda b: (b, 0)),
            pl.BlockSpec((_D, _D), lambda b: (0, 0)),
            pl.BlockSpec((blk, _D), lambda b: (b, 0)),
            pl.BlockSpec((_D, _D), lambda b: (0, 0)),
            pl.BlockSpec((2, eblk), lambda b: (0, b)),
            pl.BlockSpec((2, eblk), lambda b: (0, b)),
        ],
        out_specs=[
            pl.BlockSpec((blk, _D), lambda b: (b, 0)),
            pl.BlockSpec((blk, _D), lambda b: (b, 0)),
            pl.BlockSpec((blk, _D), lambda b: (b, 0)),
            pl.BlockSpec((eblk,), lambda b: (b,)),
            pl.BlockSpec((eblk,), lambda b: (b,)),
            pl.BlockSpec((eblk,), lambda b: (b,)),
            pl.BlockSpec((eblk,), lambda b: (b,)),
        ],
        out_shape=[
            jax.ShapeDtypeStruct((_N, _D), jnp.float32),
            jax.ShapeDtypeStruct((_N, _D), jnp.float32),
            jax.ShapeDtypeStruct((_N, _D), jnp.float32),
            jax.ShapeDtypeStruct((_E,), jnp.int32),
            jax.ShapeDtypeStruct((_E,), jnp.int32),
            jax.ShapeDtypeStruct((_E,), jnp.int32),
            jax.ShapeDtypeStruct((_E,), jnp.int32),
        ],
    )(x_user, W_u2i, x_item, W_i2u, e_u2i, e_i2u)


def _sc_body(y_u, y_i, s_u2i, d_u2i, s_i2u, d_i2u, zero_ref,
             out_item, out_user,
             sidx, didx, rows, tsidx, tdidx, trows, agg,
             sem_i, sem_g, sem_s, sem_t):
    c = lax.axis_index("c")
    s = lax.axis_index("s")
    rb = s * _RPT
    ebase = s * _EPT

    def run_rel(y_ref, src_ref, dst_ref, out_ref):
        def load_idx(k, b):
            eb = ebase + k * _C
            pltpu.async_copy(src_ref.at[pl.ds(eb, _C)], sidx[b], sem_i[b])
            pltpu.async_copy(dst_ref.at[pl.ds(eb, _C)], didx[b], sem_i[b])

        def wait_idx(b):
            pltpu.make_async_copy(
                src_ref.at[pl.ds(0, _C)], sidx[b], sem_i[b]).wait()
            pltpu.make_async_copy(
                dst_ref.at[pl.ds(0, _C)], didx[b], sem_i[b]).wait()

        def start_gather(b, g):
            pltpu.async_copy(y_ref.at[sidx[b]], rows[g], sem_g[g])

        def wait_gather(g):
            pltpu.make_async_copy(y_ref.at[sidx[0]], rows[g], sem_g[g]).wait()

        def start_scatter(g, b):
            pltpu.async_copy(rows[g], agg.at[didx[b]], sem_s[g], add=True)

        def wait_scatter(g, b):
            pltpu.make_async_copy(rows[g], agg.at[didx[b]], sem_s[g]).wait()

        for b in range(_NI):
            load_idx(b, b)
        @pl.when(s < _NS - 1)
        def _():
            pltpu.sync_copy(zero_ref.at[pl.ds(rb, _RPT)],
                            agg.at[pl.ds(rb, _RPT)])

        @pl.when(s == _NS - 1)
        def _():
            pltpu.sync_copy(zero_ref.at[pl.ds((_NS - 1) * _RPT, _N - (_NS - 1) * _RPT)],
                            agg.at[pl.ds((_NS - 1) * _RPT, _N - (_NS - 1) * _RPT)])

        plsc.subcore_barrier()
        wait_idx(0)
        start_gather(0, 0)
        wait_idx(1)
        start_gather(1, 1)
        wait_gather(0)
        start_scatter(0, 0)
        wait_idx(2)
        start_gather(2, 2)
        wait_gather(1)
        start_scatter(1, 1)
        wait_scatter(0, 0)
        load_idx(_NI, 0)
        wait_idx(3)
        start_gather(3, 0)

        def hexet(j, carry):
            for u in range(_NI):
                k = _K0 + _NI * j + u
                r = (_K0 + u) % _NR
                b = (_K0 + u) % _NI

                def guard(bound, fn):
                    th = (bound - _K0 - u) // _NI
                    if th >= _NQ - 1:
                        fn()
                    elif th >= 0:
                        pl.when(j <= th)(fn)

                guard(_NCH - 1, lambda: wait_gather(r))
                guard(_NCH - 1, lambda: start_scatter(r, b))
                guard(_NCH,
                      lambda: wait_scatter((r + 2) % _NR, (b + 5) % _NI))
                guard(_NCH - _NI,
                      lambda: load_idx(k + _NI - 1, (b + 5) % _NI))
                guard(_NCH - 3, lambda: wait_idx((b + 2) % _NI))
                guard(_NCH - 3,
                      lambda: start_gather((b + 2) % _NI, (r + 2) % _NR))

            return carry

        lax.fori_loop(0, _NQ, hexet, 0)

        et = ebase + _NCH * _C
        pltpu.async_copy(src_ref.at[pl.ds(et, _CT)], tsidx, sem_t)
        pltpu.async_copy(dst_ref.at[pl.ds(et, _CT)], tdidx, sem_t)
        pltpu.make_async_copy(src_ref.at[pl.ds(0, _CT)], tsidx, sem_t).wait()
        pltpu.make_async_copy(dst_ref.at[pl.ds(0, _CT)], tdidx, sem_t).wait()
        pltpu.async_copy(y_ref.at[tsidx], trows, sem_t).wait()
        pltpu.sync_copy(trows, agg.at[tdidx], add=True)

        plsc.subcore_barrier()
        @pl.when(s < _NS - 1)
        def _():
            pltpu.sync_copy(agg.at[pl.ds(rb, _RPT)],
                            out_ref.at[pl.ds(rb, _RPT)])

        @pl.when(s == _NS - 1)
        def _():
            pltpu.sync_copy(agg.at[pl.ds((_NS - 1) * _RPT, _N - (_NS - 1) * _RPT)],
                            out_ref.at[pl.ds((_NS - 1) * _RPT, _N - (_NS - 1) * _RPT)])

    @pl.when(c == 0)
    def _():
        run_rel(y_u, s_u2i, d_u2i, out_item)

    @pl.when(c == 1)
    def _():
        run_rel(y_i, s_i2u, d_i2u, out_user)


_sc_scatter = pl.kernel(
    _sc_body,
    out_type=(
        jax.ShapeDtypeStruct((_N, _D), jnp.float32),
        jax.ShapeDtypeStruct((_N, _D), jnp.float32),
    ),
    mesh=plsc.VectorSubcoreMesh(core_axis_name="c", subcore_axis_name="s",
                                num_cores=_NC, num_subcores=_NS),
    scratch_types=[
        [pltpu.VMEM((_C,), jnp.int32) for _ in range(_NI)],
        [pltpu.VMEM((_C,), jnp.int32) for _ in range(_NI)],
        [pltpu.VMEM((_C, _D), jnp.float32) for _ in range(_NR)],
        pltpu.VMEM((_CT,), jnp.int32),
        pltpu.VMEM((_CT,), jnp.int32),
        pltpu.VMEM((_CT, _D), jnp.float32),
        pltpu.VMEM_SHARED((_N, _D), jnp.float32),
        [pltpu.SemaphoreType.DMA for _ in range(_NI)],
        [pltpu.SemaphoreType.DMA for _ in range(_NR)],
        [pltpu.SemaphoreType.DMA for _ in range(_NR)],
        pltpu.SemaphoreType.DMA,
    ],
)


def kernel(x_user, x_item, edge_index_u2i, edge_index_i2u, W_u2i, W_i2u):
    y_user, y_item, zeros, su, du, si, di = _tc_matmul(
        x_user, W_u2i, x_item, W_i2u, edge_index_u2i, edge_index_i2u)
    out_item, out_user = _sc_scatter(
        y_user, y_item, su, du, si, di, zeros)
    return (out_user, out_item)
